# baseline (speedup 1.0000x reference)
import functools

import jax
import jax.numpy as jnp
from jax import lax
from jax.experimental import pallas as pl
from jax.experimental.pallas import tpu as pltpu
from jax.experimental.pallas import tpu_sc as plsc

HIDDEN = 768
BATCH = 4
SEQ = 2048
NC = 2
NS = 16
NW = NC * NS
TOTAL = BATCH * SEQ
RPW = TOTAL // NW
CHUNK = 32
NCHUNK = RPW // CHUNK
LANES = 16
NSLICE = HIDDEN // LANES


def _emb_body(idx_hbm, pos_hbm, tab_hbm, out_hbm, idx_v, rows_v, pos_v,
              pos_sh, gsem0, gsem1, psem, osem0, osem1):
    sid = lax.axis_index("s")
    cc = lax.axis_index("c")
    wid = sid * NC + cc
    base = wid * RPW
    pltpu.sync_copy(idx_hbm.at[wid], idx_v)

    gsem = (gsem0, gsem1)
    osem = (osem0, osem1)
    gd = [None, None]
    od = [None, None]

    def start_gather(c):
        b = c & 1
        gd[b] = pltpu.async_copy(tab_hbm.at[idx_v.at[c]], rows_v.at[b], gsem[b])

    start_gather(0)

    stage_src = (2 * (sid // 4) + cc) * 256 + lax.rem(sid, 4) * 64
    pltpu.sync_copy(pos_hbm.at[pl.ds(stage_src, 64)],
                    pos_sh.at[pl.ds(sid * 64, 64)])
    plsc.subcore_barrier()
    pos_base = lax.rem(sid, 4) * 256

    for c in range(NCHUNK):
        b = c & 1
        pd = pltpu.async_copy(
            pos_sh.at[pl.ds(pos_base + c * CHUNK, CHUNK)], pos_v, psem)
        if c + 1 < NCHUNK:
            if c >= 1:
                od[1 - b].wait()
            start_gather(c + 1)
        gd[b].wait()
        pd.wait()

        @plsc.parallel_loop(0, CHUNK, unroll=2)
        def add_row(r):
            for j in range(NSLICE):
                sl = pl.ds(j * LANES, LANES)
                plsc.addupdate(rows_v.at[b, r, sl], pos_v[r, sl])
        od[b] = pltpu.async_copy(
            rows_v.at[b], out_hbm.at[pl.ds(base + c * CHUNK, CHUNK)], osem[b])
    od[0].wait()
    od[1].wait()


@jax.jit
def _emb(idx, token_table, pos_table):
    mesh = plsc.VectorSubcoreMesh(core_axis_name="c", subcore_axis_name="s")
    f = pl.kernel(
        _emb_body,
        mesh=mesh,
        out_type=jax.ShapeDtypeStruct((TOTAL, HIDDEN), jnp.float32),
        scratch_types=[
            pltpu.VMEM((NCHUNK, CHUNK), jnp.int32),
            pltpu.VMEM((2, CHUNK, HIDDEN), jnp.float32),
            pltpu.VMEM((CHUNK, HIDDEN), jnp.float32),
            pltpu.VMEM_SHARED((4 * 256, HIDDEN), jnp.float32),
            pltpu.SemaphoreType.DMA,
            pltpu.SemaphoreType.DMA,
            pltpu.SemaphoreType.DMA,
            pltpu.SemaphoreType.DMA,
            pltpu.SemaphoreType.DMA,
        ],
    )
    return f(idx, pos_table, token_table)


def kernel(input_ids, token_table, pos_table):
    idx = input_ids.reshape(NW, NCHUNK, CHUNK).astype(jnp.int32)
    out = _emb(idx, token_table, pos_table)
    return out.reshape(BATCH, SEQ, HIDDEN)


# speedup vs baseline: 1.0777x; 1.0777x over previous
import functools

import jax
import jax.numpy as jnp
from jax import lax
from jax.experimental import pallas as pl
from jax.experimental.pallas import tpu as pltpu
from jax.experimental.pallas import tpu_sc as plsc

HIDDEN = 768
BATCH = 4
SEQ = 2048
NC = 2
NS = 16
NW = NC * NS
TOTAL = BATCH * SEQ
RPW = TOTAL // NW
CHUNK = 32
NCHUNK = RPW // CHUNK
LANES = 16
NSLICE = HIDDEN // LANES


def _emb_body(idx_hbm, pos_hbm, tab_hbm, out_hbm, idx_v, rows_v, pos_v,
              pos_sh, gsem0, gsem1, psem, osem0, osem1):
    sid = lax.axis_index("s")
    cc = lax.axis_index("c")
    wid = sid * NC + cc
    base = wid * RPW
    pltpu.sync_copy(idx_hbm.at[wid], idx_v)

    gsem = (gsem0, gsem1)
    osem = (osem0, osem1)
    gd = [None, None]
    od = [None, None]

    def start_gather(c):
        b = c & 1
        gd[b] = pltpu.async_copy(tab_hbm.at[idx_v.at[c]], rows_v.at[b], gsem[b])

    start_gather(0)

    stage_src = (2 * (sid // 4) + cc) * 256 + lax.rem(sid, 4) * 64
    pltpu.sync_copy(pos_hbm.at[pl.ds(stage_src, 64)],
                    pos_sh.at[pl.ds(sid * 64, 64)])
    plsc.subcore_barrier()
    pos_base = lax.rem(sid, 4) * 256

    for c in range(NCHUNK):
        b = c & 1
        pd = pltpu.async_copy(
            pos_sh.at[pl.ds(pos_base + c * CHUNK, CHUNK)], pos_v, psem)
        if c + 1 < NCHUNK:
            if c >= 1:
                od[1 - b].wait()
            start_gather(c + 1)
        gd[b].wait()
        pd.wait()

        def add_row(r, carry):
            for j in range(NSLICE):
                sl = pl.ds(j * LANES, LANES)
                plsc.addupdate(rows_v.at[b, r, sl], pos_v[r, sl])
            return carry

        lax.fori_loop(0, CHUNK, add_row, 0)
        od[b] = pltpu.async_copy(
            rows_v.at[b], out_hbm.at[pl.ds(base + c * CHUNK, CHUNK)], osem[b])
    od[0].wait()
    od[1].wait()


@jax.jit
def _emb(idx, token_table, pos_table):
    mesh = plsc.VectorSubcoreMesh(core_axis_name="c", subcore_axis_name="s")
    f = pl.kernel(
        _emb_body,
        mesh=mesh,
        out_type=jax.ShapeDtypeStruct((TOTAL, HIDDEN), jnp.float32),
        scratch_types=[
            pltpu.VMEM((NCHUNK, CHUNK), jnp.int32),
            pltpu.VMEM((2, CHUNK, HIDDEN), jnp.float32),
            pltpu.VMEM((CHUNK, HIDDEN), jnp.float32),
            pltpu.VMEM_SHARED((4 * 256, HIDDEN), jnp.float32),
            pltpu.SemaphoreType.DMA,
            pltpu.SemaphoreType.DMA,
            pltpu.SemaphoreType.DMA,
            pltpu.SemaphoreType.DMA,
            pltpu.SemaphoreType.DMA,
        ],
    )
    return f(idx, pos_table, token_table)


def kernel(input_ids, token_table, pos_table):
    idx = input_ids.reshape(NW, NCHUNK, CHUNK).astype(jnp.int32)
    out = _emb(idx, token_table, pos_table)
    return out.reshape(BATCH, SEQ, HIDDEN)


# speedup vs baseline: 1.1960x; 1.1098x over previous
import functools

import jax
import jax.numpy as jnp
from jax import lax
from jax.experimental import pallas as pl
from jax.experimental.pallas import tpu as pltpu
from jax.experimental.pallas import tpu_sc as plsc

HIDDEN = 768
BATCH = 4
SEQ = 2048
NC = 2
NS = 16
NW = NC * NS
SPAN = SEQ // NW
P = 8
CHUNK = P * BATCH
NCHUNK = SPAN // P
LANES = 16
NSLICE = HIDDEN // LANES


def _emb_body(idx_hbm, pos_hbm, tab_hbm, out_hbm, idx_v, rows_v, pos_v,
              gsem0, gsem1, osem0, osem1):
    sid = lax.axis_index("s")
    cc = lax.axis_index("c")
    wid = sid * NC + cc
    span0 = wid * SPAN
    pltpu.sync_copy(idx_hbm.at[wid], idx_v)

    gsem = (gsem0, gsem1)
    osem = (osem0, osem1)
    gd = [None, None]
    od = [[], []]

    def start_gather(c):
        b = c & 1
        gd[b] = pltpu.async_copy(tab_hbm.at[idx_v.at[c]], rows_v.at[b], gsem[b])

    start_gather(0)
    pltpu.sync_copy(pos_hbm.at[pl.ds(span0, SPAN)], pos_v)

    for c in range(NCHUNK):
        b = c & 1
        if c + 1 < NCHUNK:
            for dsc in od[1 - b]:
                dsc.wait()
            start_gather(c + 1)
        gd[b].wait()

        def add_pos(p, carry):
            for j in range(NSLICE):
                sl = pl.ds(j * LANES, LANES)
                x = pos_v[c * P + p, sl]
                for bt in range(BATCH):
                    plsc.addupdate(rows_v.at[b, bt * P + p, sl], x)
            return carry

        lax.fori_loop(0, P, add_pos, 0)
        od[b] = [
            pltpu.async_copy(
                rows_v.at[b, pl.ds(bt * P, P)],
                out_hbm.at[pl.ds(bt * SEQ + span0 + c * P, P)], osem[b])
            for bt in range(BATCH)
        ]
    for dsc in od[0]:
        dsc.wait()
    for dsc in od[1]:
        dsc.wait()


@jax.jit
def _emb(idx, token_table, pos_table):
    mesh = plsc.VectorSubcoreMesh(core_axis_name="c", subcore_axis_name="s")
    f = pl.kernel(
        _emb_body,
        mesh=mesh,
        out_type=jax.ShapeDtypeStruct((BATCH * SEQ, HIDDEN), jnp.float32),
        scratch_types=[
            pltpu.VMEM((NCHUNK, CHUNK), jnp.int32),
            pltpu.VMEM((2, CHUNK, HIDDEN), jnp.float32),
            pltpu.VMEM((SPAN, HIDDEN), jnp.float32),
            pltpu.SemaphoreType.DMA,
            pltpu.SemaphoreType.DMA,
            pltpu.SemaphoreType.DMA,
            pltpu.SemaphoreType.DMA,
        ],
    )
    return f(idx, pos_table, token_table)


def kernel(input_ids, token_table, pos_table):
    idx = (input_ids.astype(jnp.int32)
           .reshape(BATCH, NW, NCHUNK, P)
           .transpose(1, 2, 0, 3)
           .reshape(NW, NCHUNK, CHUNK))
    out = _emb(idx, token_table, pos_table)
    return out.reshape(BATCH, SEQ, HIDDEN)


# speedup vs baseline: 1.2230x; 1.0226x over previous
import functools

import jax
import jax.numpy as jnp
from jax import lax
from jax.experimental import pallas as pl
from jax.experimental.pallas import tpu as pltpu
from jax.experimental.pallas import tpu_sc as plsc

HIDDEN = 768
BATCH = 4
SEQ = 2048
NC = 2
NS = 16
NW = NC * NS
SPAN = SEQ // NW
P = 8
CHUNK = P * BATCH
NCHUNK = SPAN // P
NBUF = 3
LANES = 16
NSLICE = HIDDEN // LANES


def _emb_body(idx_hbm, pos_hbm, tab_hbm, out_hbm, idx_v, rows_v, pos_v,
              gsem0, gsem1, gsem2, osem0, osem1, osem2):
    sid = lax.axis_index("s")
    cc = lax.axis_index("c")
    wid = sid * NC + cc
    span0 = wid * SPAN
    pltpu.sync_copy(idx_hbm.at[wid], idx_v)

    gsem = (gsem0, gsem1, gsem2)
    osem = (osem0, osem1, osem2)
    gd = [None] * NBUF
    od = [[] for _ in range(NBUF)]

    def start_gather(c):
        b = c % NBUF
        gd[b] = pltpu.async_copy(tab_hbm.at[idx_v.at[c]], rows_v.at[b], gsem[b])

    start_gather(0)
    start_gather(1)
    pltpu.sync_copy(pos_hbm.at[pl.ds(span0, SPAN)], pos_v)

    for c in range(NCHUNK):
        b = c % NBUF
        gd[b].wait()
        if c + 2 < NCHUNK:
            b2 = (c + 2) % NBUF
            for dsc in od[b2]:
                dsc.wait()
            start_gather(c + 2)

        def add_pos(p, carry):
            for j in range(NSLICE):
                sl = pl.ds(j * LANES, LANES)
                x = pos_v[c * P + p, sl]
                for bt in range(BATCH):
                    plsc.addupdate(rows_v.at[b, bt * P + p, sl], x)
            return carry

        lax.fori_loop(0, P, add_pos, 0)
        od[b] = [
            pltpu.async_copy(
                rows_v.at[b, pl.ds(bt * P, P)],
                out_hbm.at[pl.ds(bt * SEQ + span0 + c * P, P)], osem[b])
            for bt in range(BATCH)
        ]
    for lst in od:
        for dsc in lst:
            dsc.wait()


@jax.jit
def _emb(idx, token_table, pos_table):
    mesh = plsc.VectorSubcoreMesh(core_axis_name="c", subcore_axis_name="s")
    f = pl.kernel(
        _emb_body,
        mesh=mesh,
        out_type=jax.ShapeDtypeStruct((BATCH * SEQ, HIDDEN), jnp.float32),
        scratch_types=[
            pltpu.VMEM((NCHUNK, CHUNK), jnp.int32),
            pltpu.VMEM((NBUF, CHUNK, HIDDEN), jnp.float32),
            pltpu.VMEM((SPAN, HIDDEN), jnp.float32),
            pltpu.SemaphoreType.DMA,
            pltpu.SemaphoreType.DMA,
            pltpu.SemaphoreType.DMA,
            pltpu.SemaphoreType.DMA,
            pltpu.SemaphoreType.DMA,
            pltpu.SemaphoreType.DMA,
        ],
    )
    return f(idx, pos_table, token_table)


def kernel(input_ids, token_table, pos_table):
    idx = (input_ids.astype(jnp.int32)
           .reshape(BATCH, NW, NCHUNK, P)
           .transpose(1, 2, 0, 3)
           .reshape(NW, NCHUNK, CHUNK))
    out = _emb(idx, token_table, pos_table)
    return out.reshape(BATCH, SEQ, HIDDEN)
